# Initial kernel scaffold; baseline (speedup 1.0000x reference)
import jax
import jax.numpy as jnp
from jax.experimental import pallas as pl


def kernel(feature, edge_index, W, b):
    raise NotImplementedError("write your pallas kernel here")


# speedup vs baseline: 35.6170x; 35.6170x over previous
import functools

import jax
import jax.numpy as jnp
from jax import lax
from jax.experimental import pallas as pl
from jax.experimental.pallas import tpu as pltpu
from jax.experimental.pallas import tpu_sc as plsc

N = 10000
E = 320000
D = 128
K = 5000
NT = 16
N2 = 10240
CHN = N2 // NT
CHE = E // NT
RB = 256
L = 16
EROWS = 160
PROWS = CHN // 128
MASK_SUB = 2000
ROW_SUB = 40
PAD_NODE = N2 - 1
T7 = K - 7 * CHN


def _mm_kernel(f_ref, w_ref, o_ref):
    o_ref[...] = jnp.dot(f_ref[...], w_ref[...],
                         preferred_element_type=jnp.float32)


def _desc_key(s):
    u = plsc.bitcast(s, jnp.int32)
    m = jnp.where(u < 0, ~u, u ^ jnp.int32(-2147483648))
    return ~m


def _key_to_score(k):
    m = ~k
    u = jnp.where(m < 0, m ^ jnp.int32(-2147483648), ~m)
    return plsc.bitcast(u, jnp.float32)


def _tanh16(x):
    a = jnp.abs(x)
    e = jnp.exp(a * jnp.float32(-2.0))
    t = (jnp.float32(1.0) - e) / (jnp.float32(1.0) + e)
    return jnp.where(x < 0, -t, t)


def _sc_body(feat_hbm, src_hbm, dst_hbm, msg_hbm, b_hbm,
             feat_out, perm_out, mask_out,
             msg_v, srcv, dstv, valv, keepv, scorev, keysv, valsv,
             histv, offv, tablev, onesv, maskv, rowsv, tb_v, b_v,
             dst2, pos2, idx2, perm520,
             keys_sA, vals_sA, keys_sB, vals_sB, score_s, cnt_s, keep_s,
             sem):
    cid = lax.axis_index("c")
    tid = lax.axis_index("s")

    @pl.when(cid == 0)
    def _core0():
        base_e = tid * CHE
        base_n = tid * CHN
        zf = jnp.zeros((L,), jnp.float32)
        zi = jnp.zeros((L,), jnp.int32)
        oi = jnp.ones((L,), jnp.int32)
        lane = lax.broadcasted_iota(jnp.int32, (L,), 0)
        lane0 = lane == 0

        pltpu.sync_copy(msg_hbm, msg_v)
        pltpu.sync_copy(src_hbm.at[pl.ds(base_e, CHE)], srcv.at[pl.ds(0, CHE)])
        pltpu.sync_copy(dst_hbm.at[pl.ds(base_e, CHE)], dstv.at[pl.ds(0, CHE)])
        pltpu.sync_copy(b_hbm, b_v)

        def _zf(i, _):
            valv[pl.ds(i * L, L)] = zf
            return 0

        lax.fori_loop(0, N2 // L, _zf, 0)

        def _zk(i, _):
            keepv[pl.ds(i * L, L)] = zi
            return 0

        lax.fori_loop(0, N2 // L, _zk, 0)

        @pl.when(tid == 0)
        def _():
            pltpu.sync_copy(valv.at[pl.ds(0, N2)], score_s)
            pltpu.sync_copy(keepv, keep_s)

        plsc.subcore_barrier()

        def _gath(i, _):
            idx = srcv[pl.ds(i * L, L)]
            valv[pl.ds(i * L, L)] = plsc.load_gather(msg_v, [idx])
            return 0

        lax.fori_loop(0, CHE // L, _gath, 0)
        for t in range((EROWS * 128 - CHE) // L):
            dstv[pl.ds(CHE + t * L, L)] = zi + PAD_NODE
            valv[pl.ds(CHE + t * L, L)] = zf

        def _d2(j, _):
            for c in range(128 // L):
                dst2[j, pl.ds(c * L, L)] = dstv[pl.ds(j * 128 + c * L, L)]
            return 0

        lax.fori_loop(0, EROWS, _d2, 0)

        def _scadd(g, _):
            hs = []
            for u in range(8):
                j = g * 8 + u
                hs.append(pltpu.async_copy(
                    valv.at[pl.ds(j * 128, 128)],
                    score_s.at[dst2.at[j]], sem, add=True))
            for h in hs:
                h.wait()
            return 0

        lax.fori_loop(0, EROWS // 8, _scadd, 0)
        plsc.subcore_barrier()

        pltpu.sync_copy(score_s.at[pl.ds(base_n, CHN)], scorev)
        bvec = b_v[...]

        def _mkkey(i, _):
            s = scorev[pl.ds(i * L, L)] + bvec
            gidx = base_n + i * L + lane
            kd = _desc_key(s)
            keysv[pl.ds(i * L, L)] = jnp.where(gidx < N, kd, jnp.int32(-1))
            valsv[pl.ds(i * L, L)] = gidx
            return 0

        lax.fori_loop(0, CHN // L, _mkkey, 0)
        pltpu.sync_copy(keysv, keys_sA.at[pl.ds(base_n, CHN)])
        pltpu.sync_copy(valsv, vals_sA.at[pl.ds(base_n, CHN)])
        plsc.subcore_barrier()

        bufs = [(keys_sA, vals_sA, keys_sB, vals_sB),
                (keys_sB, vals_sB, keys_sA, vals_sA),
                (keys_sA, vals_sA, keys_sB, vals_sB),
                (keys_sB, vals_sB, keys_sA, vals_sA)]
        for p in range(4):
            shift = 8 * p
            kin, vin, kout, vout = bufs[p]
            pltpu.sync_copy(kin.at[pl.ds(base_n, CHN)], keysv)
            pltpu.sync_copy(vin.at[pl.ds(base_n, CHN)], valsv)

            def _zh(i, _):
                histv[pl.ds(i * L, L)] = zi
                return 0

            lax.fori_loop(0, RB // L, _zh, 0)

            def _hist(i, _):
                kv = keysv[pl.ds(i * L, L)]
                dv = lax.shift_right_logical(kv, shift) & 255
                for l in range(L):
                    db = dv[l] + zi
                    cur = plsc.load_gather(histv, [db])
                    plsc.store_scatter(histv, [db], cur + 1, mask=lane0)
                return 0

            lax.fori_loop(0, CHN // L, _hist, 0)
            pltpu.sync_copy(histv, cnt_s.at[pl.ds(tid * RB, RB)])
            plsc.subcore_barrier()
            pltpu.sync_copy(cnt_s, tablev)

            carry = jnp.int32(0)
            for c in range(RB // L):
                tot = zi
                for t in range(NT):
                    tot = tot + tablev[pl.ds(t * RB + c * L, L)]
                incl = plsc.cumsum(tot)
                excl = incl - tot

                def _ptloop(t, acc):
                    return acc + tablev[pl.ds(t * RB + c * L, L)]

                me = lax.fori_loop(0, tid, _ptloop, zi)
                offv[pl.ds(c * L, L)] = excl + me + carry
                carry = carry + jnp.sum(tot)

            def _scat(i, _):
                kv = keysv[pl.ds(i * L, L)]
                dv = lax.shift_right_logical(kv, shift) & 255
                pos16 = zi
                for l in range(L):
                    db = dv[l] + zi
                    cur = plsc.load_gather(offv, [db])
                    plsc.store_scatter(offv, [db], cur + 1, mask=lane0)
                    pos16 = jnp.where(lane == l, cur[0], pos16)
                row = i // (128 // L)
                col = i % (128 // L)
                pos2[row, pl.ds(col * L, L)] = pos16
                return 0

            lax.fori_loop(0, CHN // L, _scat, 0)
            for j in range(PROWS):
                pltpu.sync_copy(keysv.at[pl.ds(j * 128, 128)],
                                kout.at[pos2.at[j]])
                pltpu.sync_copy(valsv.at[pl.ds(j * 128, 128)],
                                vout.at[pos2.at[j]])
            plsc.subcore_barrier()

        kfin, vfin = keys_sA, vals_sA

        pltpu.sync_copy(vfin.at[pl.ds(base_n, CHN)], valsv)
        pltpu.sync_copy(kfin.at[pl.ds(base_n, CHN)], keysv)

        def _ones(i, _):
            onesv[pl.ds(i * L, L)] = oi
            return 0

        lax.fori_loop(0, CHN // L, _ones, 0)

        def _i2(j, _):
            for c in range(128 // L):
                idx2[j, pl.ds(c * L, L)] = valsv[pl.ds(j * 128 + c * L, L)]
            return 0

        lax.fori_loop(0, PROWS, _i2, 0)

        @pl.when(tid < 7)
        def _():
            pltpu.sync_copy(valsv, perm_out.at[pl.ds(base_n, CHN)])
            for j in range(PROWS):
                pltpu.sync_copy(onesv.at[pl.ds(0, 128)],
                                keep_s.at[idx2.at[j]])

        @pl.when(tid == 7)
        def _():
            for c in range(T7 // L):
                perm520[pl.ds(c * L, L)] = valsv[pl.ds(c * L, L)]
            perm520[pl.ds(T7 - L, L)] = valsv[pl.ds(T7 - L, L)]
            for j in range(PROWS):
                for c in range(128 // L):
                    g = j * 128 + c * L
                    cur = idx2[j, pl.ds(c * L, L)]
                    idx2[j, pl.ds(c * L, L)] = jnp.where(
                        g + lane < T7, cur, jnp.int32(PAD_NODE))
            pltpu.sync_copy(perm520, perm_out.at[pl.ds(7 * CHN, T7)])
            for j in range(PROWS):
                pltpu.sync_copy(onesv.at[pl.ds(0, 128)],
                                keep_s.at[idx2.at[j]])

        plsc.subcore_barrier()

        pltpu.sync_copy(keep_s, keepv)
        for j in range(CHE // MASK_SUB):
            def _em(i, _):
                off = j * MASK_SUB + i * L
                ks = plsc.load_gather(keepv, [srcv[pl.ds(off, L)]])
                kd = plsc.load_gather(keepv, [dstv[pl.ds(off, L)]])
                maskv[pl.ds(i * L, L)] = ks & kd
                return 0

            lax.fori_loop(0, MASK_SUB // L, _em, 0)
            pltpu.sync_copy(
                maskv, mask_out.at[pl.ds(base_e + j * MASK_SUB, MASK_SUB)])

        @pl.when(tid < 8)
        def _rows():
            def _tb(i, _):
                sc = _key_to_score(keysv[pl.ds(i * L, L)])
                tb_v[pl.ds(i * L, L)] = _tanh16(sc)
                return 0

            lax.fori_loop(0, CHN // L, _tb, 0)
            nbat = jnp.where(tid < 7, CHN // ROW_SUB, T7 // ROW_SUB)

            def _batch(g, _):
                rbase = g * ROW_SUB
                pltpu.async_copy(
                    feat_hbm.at[valsv.at[pl.ds(rbase, ROW_SUB)]],
                    rowsv, sem).wait()

                def _scale(r, _):
                    tvec = tb_v[pl.ds(rbase + r, L)]
                    t0 = tvec[0]
                    tv = t0 * jnp.ones((L,), jnp.float32)
                    for c in range(D // L):
                        rowsv[r, pl.ds(c * L, L)] = (
                            rowsv[r, pl.ds(c * L, L)] * tv)
                    return 0

                lax.fori_loop(0, ROW_SUB, _scale, 0)
                pltpu.sync_copy(
                    rowsv, feat_out.at[pl.ds(base_n + rbase, ROW_SUB)])
                return 0

            lax.fori_loop(0, nbat, _batch, 0)


@jax.jit
def kernel(feature, edge_index, W, b):
    msg = pl.pallas_call(
        _mm_kernel,
        out_shape=jax.ShapeDtypeStruct((N, 1), jnp.float32))(feature, W)
    msg1 = msg.reshape(N)
    b16 = jnp.broadcast_to(b.astype(jnp.float32), (L,))

    mesh = plsc.VectorSubcoreMesh(core_axis_name="c", subcore_axis_name="s")
    sck = functools.partial(
        pl.kernel, mesh=mesh,
        compiler_params=pltpu.CompilerParams(use_tc_tiling_on_sc=False, needs_layout_passes=False),
        out_type=[jax.ShapeDtypeStruct((K, D), jnp.float32),
                  jax.ShapeDtypeStruct((K,), jnp.int32),
                  jax.ShapeDtypeStruct((E,), jnp.int32)],
        scratch_types=[
            pltpu.VMEM((N,), jnp.float32),
            pltpu.VMEM((EROWS * 128,), jnp.int32),
            pltpu.VMEM((EROWS * 128,), jnp.int32),
            pltpu.VMEM((EROWS * 128,), jnp.float32),
            pltpu.VMEM((N2,), jnp.int32),
            pltpu.VMEM((CHN,), jnp.float32),
            pltpu.VMEM((CHN,), jnp.int32),
            pltpu.VMEM((CHN,), jnp.int32),
            pltpu.VMEM((RB,), jnp.int32),
            pltpu.VMEM((RB,), jnp.int32),
            pltpu.VMEM((RB * NT,), jnp.int32),
            pltpu.VMEM((CHN,), jnp.int32),
            pltpu.VMEM((MASK_SUB,), jnp.int32),
            pltpu.VMEM((ROW_SUB, D), jnp.float32),
            pltpu.VMEM((CHN + L,), jnp.float32),
            pltpu.VMEM((L,), jnp.float32),
            pltpu.VMEM((EROWS, 128), jnp.int32),
            pltpu.VMEM((PROWS, 128), jnp.int32),
            pltpu.VMEM((PROWS, 128), jnp.int32),
            pltpu.VMEM((T7,), jnp.int32),
            pltpu.VMEM_SHARED((N2,), jnp.int32),
            pltpu.VMEM_SHARED((N2,), jnp.int32),
            pltpu.VMEM_SHARED((N2,), jnp.int32),
            pltpu.VMEM_SHARED((N2,), jnp.int32),
            pltpu.VMEM_SHARED((N2,), jnp.float32),
            pltpu.VMEM_SHARED((RB * NT,), jnp.int32),
            pltpu.VMEM_SHARED((N2,), jnp.int32),
            pltpu.SemaphoreType.DMA,
        ])(_sc_body)
    feat_out, perm, maski = sck(feature, edge_index[0], edge_index[1], msg1, b16)
    return feat_out, perm, maski.astype(jnp.bool_)
